# Initial kernel scaffold; baseline (speedup 1.0000x reference)
import jax
import jax.numpy as jnp
from jax.experimental import pallas as pl


def kernel(x, table, pos):
    raise NotImplementedError("write your pallas kernel here")


# speedup vs baseline: 2.5461x; 2.5461x over previous
import jax
import jax.numpy as jnp
from jax import lax
from jax.experimental import pallas as pl
from jax.experimental.pallas import tpu as pltpu
from jax.experimental.pallas import tpu_sc as plsc

VOCAB = 100000
D_MODEL = 2048
B, S = 4, 2048
N_FLAT = B * S

NC, NS = 2, 16
NW = NC * NS
PER_W = N_FLAT // NW
CHUNK = 16
N_CHUNK = PER_W // CHUNK
LANES = 16
VECS_PER_ROW = D_MODEL // LANES


def _body(x_hbm, table_hbm, pos_hbm, out_hbm, idx_v, buf, posbuf, sem):
    wid = lax.axis_index("s") * NC + lax.axis_index("c")
    base = wid * PER_W
    sbase = base % S

    pltpu.sync_copy(x_hbm.at[pl.ds(base, PER_W)], idx_v)

    def chunk_body(c, _):
        gather = pltpu.async_copy(
            table_hbm.at[idx_v.at[pl.ds(c * CHUNK, CHUNK)]], buf, sem
        )
        pltpu.sync_copy(pos_hbm.at[pl.ds(sbase + c * CHUNK, CHUNK)], posbuf)
        gather.wait()

        def row_body(i, _):
            def vec_body(j, _):
                sl = pl.ds(j * LANES, LANES)
                buf[i, sl] = buf[i, sl] + posbuf[i, sl]
                return 0

            lax.fori_loop(0, VECS_PER_ROW, vec_body, 0)
            return 0

        lax.fori_loop(0, CHUNK, row_body, 0)
        pltpu.sync_copy(buf, out_hbm.at[pl.ds(base + c * CHUNK, CHUNK)])
        return 0

    lax.fori_loop(0, N_CHUNK, chunk_body, 0)


@jax.jit
def kernel(x, table, pos):
    x_flat = x.reshape(N_FLAT).astype(jnp.int32)
    mesh = plsc.VectorSubcoreMesh(core_axis_name="c", subcore_axis_name="s")
    out = pl.kernel(
        _body,
        out_type=jax.ShapeDtypeStruct((N_FLAT, D_MODEL), jnp.float32),
        mesh=mesh,
        scratch_types=[
            pltpu.VMEM((PER_W,), jnp.int32),
            pltpu.VMEM((CHUNK, D_MODEL), jnp.float32),
            pltpu.VMEM((CHUNK, D_MODEL), jnp.float32),
            pltpu.SemaphoreType.DMA,
        ],
    )(x_flat, table, pos)
    return out.reshape(B, S, D_MODEL)


# speedup vs baseline: 4.2483x; 1.6686x over previous
import jax
import jax.numpy as jnp
from jax import lax
from jax.experimental import pallas as pl
from jax.experimental.pallas import tpu as pltpu
from jax.experimental.pallas import tpu_sc as plsc

VOCAB = 100000
D_MODEL = 2048
B, S = 4, 2048
N_FLAT = B * S

NC, NS = 2, 16
NW = NC * NS
PER_W = N_FLAT // NW
CHUNK = 8
N_CHUNK = PER_W // CHUNK
LANES = 16
VECS_PER_ROW = D_MODEL // LANES


def _body(
    x_hbm,
    table_hbm,
    pos_hbm,
    out_hbm,
    idx_v,
    buf0,
    buf1,
    pos0,
    pos1,
    sg0,
    sg1,
    sp0,
    sp1,
    ss0,
    ss1,
):
    wid = lax.axis_index("s") * NC + lax.axis_index("c")
    base = wid * PER_W
    sbase = base % S

    bufs = (buf0, buf1)
    poss = (pos0, pos1)
    sgs = (sg0, sg1)
    sps = (sp0, sp1)
    sss = (ss0, ss1)

    pltpu.sync_copy(x_hbm.at[pl.ds(base, PER_W)], idx_v)

    def issue_loads(c, slot):
        pltpu.async_copy(
            table_hbm.at[idx_v.at[pl.ds(c * CHUNK, CHUNK)]], bufs[slot], sgs[slot]
        )
        pltpu.async_copy(
            pos_hbm.at[pl.ds(sbase + c * CHUNK, CHUNK)], poss[slot], sps[slot]
        )

    def wait_loads(slot):
        pltpu.make_async_copy(table_hbm.at[pl.ds(0, CHUNK)], bufs[slot], sgs[slot]).wait()
        pltpu.make_async_copy(pos_hbm.at[pl.ds(0, CHUNK)], poss[slot], sps[slot]).wait()

    def wait_store(slot):
        pltpu.make_async_copy(bufs[slot], out_hbm.at[pl.ds(0, CHUNK)], sss[slot]).wait()

    def do_chunk(c, slot):
        other = 1 - slot

        @pl.when(c + 1 < N_CHUNK)
        def _():
            @pl.when(c >= 1)
            def _():
                wait_store(other)

            issue_loads(c + 1, other)

        wait_loads(slot)

        buf, posb = bufs[slot], poss[slot]

        def row_body(i, _):
            for j in range(VECS_PER_ROW):
                sl = pl.ds(j * LANES, LANES)
                buf[i, sl] = buf[i, sl] + posb[i, sl]
            return 0

        lax.fori_loop(0, CHUNK, row_body, 0)
        pltpu.async_copy(buf, out_hbm.at[pl.ds(base + c * CHUNK, CHUNK)], sss[slot])

    issue_loads(0, 0)

    def pair(cc, _):
        do_chunk(cc, 0)
        do_chunk(cc + 1, 1)
        return 0

    lax.fori_loop(0, N_CHUNK // 2, lambda k, _: pair(k * 2, _), 0)

    wait_store(0)
    wait_store(1)


@jax.jit
def kernel(x, table, pos):
    x_flat = x.reshape(N_FLAT).astype(jnp.int32)
    mesh = plsc.VectorSubcoreMesh(core_axis_name="c", subcore_axis_name="s")
    out = pl.kernel(
        _body,
        out_type=jax.ShapeDtypeStruct((N_FLAT, D_MODEL), jnp.float32),
        mesh=mesh,
        scratch_types=[
            pltpu.VMEM((PER_W,), jnp.int32),
            pltpu.VMEM((CHUNK, D_MODEL), jnp.float32),
            pltpu.VMEM((CHUNK, D_MODEL), jnp.float32),
            pltpu.VMEM((CHUNK, D_MODEL), jnp.float32),
            pltpu.VMEM((CHUNK, D_MODEL), jnp.float32),
            pltpu.SemaphoreType.DMA,
            pltpu.SemaphoreType.DMA,
            pltpu.SemaphoreType.DMA,
            pltpu.SemaphoreType.DMA,
            pltpu.SemaphoreType.DMA,
            pltpu.SemaphoreType.DMA,
        ],
    )(x_flat, table, pos)
    return out.reshape(B, S, D_MODEL)


# speedup vs baseline: 6.2746x; 1.4770x over previous
import jax
import jax.numpy as jnp
from jax import lax
from jax.experimental import pallas as pl
from jax.experimental.pallas import tpu as pltpu
from jax.experimental.pallas import tpu_sc as plsc

VOCAB = 100000
D_MODEL = 2048
B, S = 4, 2048
N_FLAT = B * S

NC, NS = 2, 16
NW = NC * NS
S_PER_W = S // NW
C_S = 4
ROWS = B * C_S
N_CHUNK = S_PER_W // C_S
PER_W = B * S_PER_W
LANES = 16
VECS_PER_ROW = D_MODEL // LANES


def _body(
    x_hbm,
    table_hbm,
    pos_hbm,
    out_hbm,
    idx_v,
    buf0,
    buf1,
    pos0,
    pos1,
    sg0,
    sg1,
    sp0,
    sp1,
    ss0,
    ss1,
):
    wid = lax.axis_index("s") * NC + lax.axis_index("c")
    sbase = wid * S_PER_W

    bufs = (buf0, buf1)
    poss = (pos0, pos1)
    sgs = (sg0, sg1)
    sps = (sp0, sp1)
    sss = (ss0, ss1)

    pltpu.sync_copy(x_hbm.at[pl.ds(wid * PER_W, PER_W)], idx_v)

    def issue_loads(c, slot):
        pltpu.async_copy(
            table_hbm.at[idx_v.at[pl.ds(c * ROWS, ROWS)]], bufs[slot], sgs[slot]
        )
        pltpu.async_copy(
            pos_hbm.at[pl.ds(sbase + c * C_S, C_S)], poss[slot], sps[slot]
        )

    def wait_loads(slot):
        pltpu.make_async_copy(table_hbm.at[pl.ds(0, ROWS)], bufs[slot], sgs[slot]).wait()
        pltpu.make_async_copy(pos_hbm.at[pl.ds(0, C_S)], poss[slot], sps[slot]).wait()

    def wait_store(slot):
        pltpu.make_async_copy(bufs[slot], out_hbm.at[pl.ds(0, ROWS)], sss[slot]).wait()

    def do_chunk(c, slot):
        other = 1 - slot

        @pl.when(c + 1 < N_CHUNK)
        def _():
            @pl.when(c >= 1)
            def _():
                wait_store(other)

            issue_loads(c + 1, other)

        wait_loads(slot)

        buf, posb = bufs[slot], poss[slot]

        def vec_body(j, _):
            sl = pl.ds(j * LANES, LANES)
            for t in range(C_S):
                pv = posb[t, sl]
                for b in range(B):
                    r = b * C_S + t
                    buf[r, sl] = buf[r, sl] + pv
            return 0

        lax.fori_loop(0, VECS_PER_ROW, vec_body, 0)

        for b in range(B):
            pltpu.async_copy(
                buf.at[pl.ds(b * C_S, C_S)],
                out_hbm.at[pl.ds(b * S + sbase + c * C_S, C_S)],
                sss[slot],
            )

    issue_loads(0, 0)

    def pair(k, _):
        do_chunk(k * 2, 0)
        do_chunk(k * 2 + 1, 1)
        return 0

    lax.fori_loop(0, N_CHUNK // 2, pair, 0)

    wait_store(0)
    wait_store(1)


@jax.jit
def kernel(x, table, pos):
    xr = (
        x.reshape(B, NW, N_CHUNK, C_S)
        .transpose(1, 2, 0, 3)
        .reshape(N_FLAT)
        .astype(jnp.int32)
    )
    mesh = plsc.VectorSubcoreMesh(core_axis_name="c", subcore_axis_name="s")
    out = pl.kernel(
        _body,
        out_type=jax.ShapeDtypeStruct((N_FLAT, D_MODEL), jnp.float32),
        mesh=mesh,
        scratch_types=[
            pltpu.VMEM((PER_W,), jnp.int32),
            pltpu.VMEM((ROWS, D_MODEL), jnp.float32),
            pltpu.VMEM((ROWS, D_MODEL), jnp.float32),
            pltpu.VMEM((C_S, D_MODEL), jnp.float32),
            pltpu.VMEM((C_S, D_MODEL), jnp.float32),
            pltpu.SemaphoreType.DMA,
            pltpu.SemaphoreType.DMA,
            pltpu.SemaphoreType.DMA,
            pltpu.SemaphoreType.DMA,
            pltpu.SemaphoreType.DMA,
            pltpu.SemaphoreType.DMA,
        ],
    )(xr, table, pos)
    return out.reshape(B, S, D_MODEL)
